# baseline (device time: 169852 ns/iter reference)
import jax
import jax.numpy as jnp
from jax import lax
from jax.experimental import pallas as pl
from jax.experimental.pallas import tpu as pltpu

P = 32


def kernel(A, B):
    m, k = A.shape
    _, n = B.shape
    mc = m // P

    def body(a_ref, b_ref, out_ref, acc_ref, sbuf_ref, comm_ref,
             send_sems, recv_sems):
        my = lax.axis_index("i")
        left = lax.rem(my - 1 + P, P)
        right = lax.rem(my + 1, P)

        barrier_sem = pltpu.get_barrier_semaphore()
        for nbr in (left, right):
            pl.semaphore_signal(
                barrier_sem, inc=1,
                device_id=(nbr,), device_id_type=pl.DeviceIdType.MESH,
            )
        pl.semaphore_wait(barrier_sem, 2)

        acc_ref[...] = jnp.dot(
            a_ref[...].astype(jnp.bfloat16),
            b_ref[...].astype(jnp.bfloat16),
            preferred_element_type=jnp.float32,
        )

        for s in range(P - 1):
            c = lax.rem(my - s - 1 + P, P)
            chunk = acc_ref[pl.ds(c * mc, mc), :]
            if s == 0:
                sbuf_ref[s % 2, :, :] = chunk
            else:
                sbuf_ref[s % 2, :, :] = chunk + comm_ref[s - 1, :, :]
            rdma = pltpu.make_async_remote_copy(
                src_ref=sbuf_ref.at[s % 2],
                dst_ref=comm_ref.at[s],
                send_sem=send_sems.at[s],
                recv_sem=recv_sems.at[s],
                device_id=(right,),
                device_id_type=pl.DeviceIdType.MESH,
            )
            rdma.start()
            rdma.wait()

        out_ref[...] = acc_ref[pl.ds(my * mc, mc), :] + comm_ref[P - 2, :, :]

    return pl.pallas_call(
        body,
        out_shape=jax.ShapeDtypeStruct((mc, n), jnp.float32),
        in_specs=[
            pl.BlockSpec(memory_space=pltpu.VMEM),
            pl.BlockSpec(memory_space=pltpu.VMEM),
        ],
        out_specs=pl.BlockSpec(memory_space=pltpu.VMEM),
        scratch_shapes=[
            pltpu.VMEM((m, n), jnp.float32),
            pltpu.VMEM((2, mc, n), jnp.float32),
            pltpu.VMEM((P - 1, mc, n), jnp.float32),
            pltpu.SemaphoreType.DMA((P - 1,)),
            pltpu.SemaphoreType.DMA((P - 1,)),
        ],
        compiler_params=pltpu.CompilerParams(collective_id=0),
    )(A, B)


# device time: 120760 ns/iter; 1.4065x vs baseline; 1.4065x over previous
import jax
import jax.numpy as jnp
from jax import lax
from jax.experimental import pallas as pl
from jax.experimental.pallas import tpu as pltpu

P = 32


def kernel(A, B):
    m, k = A.shape
    _, n = B.shape
    mc = m // P

    def body(a_ref, b_ref, out_ref, acc_ref, sbuf_ref, comm_ref,
             send_sems, recv_sems):
        my = lax.axis_index("i")
        left = lax.rem(my - 1 + P, P)
        right = lax.rem(my + 1, P)

        barrier_sem = pltpu.get_barrier_semaphore()
        for nbr in (left, right):
            pl.semaphore_signal(
                barrier_sem, inc=1,
                device_id=(nbr,), device_id_type=pl.DeviceIdType.MESH,
            )
        pl.semaphore_wait(barrier_sem, 2)

        acc_ref[...] = jnp.dot(
            a_ref[...].astype(jnp.bfloat16),
            b_ref[...].astype(jnp.bfloat16),
            preferred_element_type=jnp.float32,
        )

        for s in range(P - 1):
            c = lax.rem(my - s - 1 + P, P)
            chunk = acc_ref[pl.ds(c * mc, mc), :]
            if s == 0:
                val = chunk
            else:
                val = chunk + comm_ref[s - 1, :, :]
            sbuf_ref[s % 2, :, :] = val.astype(jnp.bfloat16)
            rdma = pltpu.make_async_remote_copy(
                src_ref=sbuf_ref.at[s % 2],
                dst_ref=comm_ref.at[s],
                send_sem=send_sems.at[s],
                recv_sem=recv_sems.at[s],
                device_id=(right,),
                device_id_type=pl.DeviceIdType.MESH,
            )
            rdma.start()
            rdma.wait()

        out_ref[...] = acc_ref[pl.ds(my * mc, mc), :] + comm_ref[P - 2, :, :]

    return pl.pallas_call(
        body,
        out_shape=jax.ShapeDtypeStruct((mc, n), jnp.float32),
        in_specs=[
            pl.BlockSpec(memory_space=pltpu.VMEM),
            pl.BlockSpec(memory_space=pltpu.VMEM),
        ],
        out_specs=pl.BlockSpec(memory_space=pltpu.VMEM),
        scratch_shapes=[
            pltpu.VMEM((m, n), jnp.float32),
            pltpu.VMEM((2, mc, n), jnp.bfloat16),
            pltpu.VMEM((P - 1, mc, n), jnp.bfloat16),
            pltpu.SemaphoreType.DMA((P - 1,)),
            pltpu.SemaphoreType.DMA((P - 1,)),
        ],
        compiler_params=pltpu.CompilerParams(collective_id=0),
    )(A, B)


# device time: 77742 ns/iter; 2.1848x vs baseline; 1.5533x over previous
import jax
import jax.numpy as jnp
from jax import lax
from jax.experimental import pallas as pl
from jax.experimental.pallas import tpu as pltpu

P = 32
NST = 5
BITS = (0, 3, 1, 2, 4)
SZ = (16, 8, 4, 2, 1)
OFF = (0, 16, 24, 28, 30)


def _slot_of(c: int) -> int:
    s = 0
    for i, b in enumerate(BITS):
        s |= ((c >> b) & 1) << (NST - 1 - i)
    return s


def kernel(A, B):
    m, k = A.shape
    _, n = B.shape
    mc = m // P

    def body(a_ref, b_ref, out_ref, aperm_ref, bbf_ref, acc_ref,
             sbuf_ref, rbuf_ref, send_sems, recv_sems):
        my = lax.axis_index("i")

        myslot = jnp.int32(0)
        for i, b in enumerate(BITS):
            myslot = myslot | (((my >> b) & 1) << (NST - 1 - i))

        partners = [my ^ (1 << b) for b in BITS]

        barrier_sem = pltpu.get_barrier_semaphore()
        for pr in partners:
            pl.semaphore_signal(
                barrier_sem, inc=1,
                device_id=(pr,), device_id_type=pl.DeviceIdType.MESH,
            )
        pl.semaphore_wait(barrier_sem, NST)

        bbf_ref[...] = b_ref[...].astype(jnp.bfloat16)
        for c in range(P):
            sl = _slot_of(c)
            aperm_ref[sl * mc:(sl + 1) * mc, :] = (
                a_ref[c * mc:(c + 1) * mc, :].astype(jnp.bfloat16))

        ab = jnp.int32(0)
        kept_bases = []
        acc_base = None
        for s in range(NST):
            sz = SZ[s]
            ms = (myslot >> (NST - 1 - s)) & 1
            send_base = ab + (1 - ms) * sz
            kept_base = ab + ms * sz

            if s == 0:
                val = jnp.dot(
                    aperm_ref[pl.ds(send_base * mc, sz * mc), :],
                    bbf_ref[...], preferred_element_type=jnp.float32)
            else:
                val = acc_ref[pl.ds((send_base - acc_base) * mc, sz * mc), :]
                for t in range(s):
                    off = send_base - kept_bases[t]
                    val = val + rbuf_ref[
                        pl.ds((OFF[t] + off) * mc, sz * mc), :
                    ].astype(jnp.float32)
            sbuf_ref[OFF[s] * mc:(OFF[s] + sz) * mc, :] = (
                val.astype(jnp.bfloat16))

            rdma = pltpu.make_async_remote_copy(
                src_ref=sbuf_ref.at[pl.ds(OFF[s] * mc, sz * mc), :],
                dst_ref=rbuf_ref.at[pl.ds(OFF[s] * mc, sz * mc), :],
                send_sem=send_sems.at[s],
                recv_sem=recv_sems.at[s],
                device_id=(partners[s],),
                device_id_type=pl.DeviceIdType.MESH,
            )
            rdma.start()

            if s == 0:
                acc_base = kept_base
                acc_ref[...] = jnp.dot(
                    aperm_ref[pl.ds(kept_base * mc, (P // 2) * mc), :],
                    bbf_ref[...], preferred_element_type=jnp.float32)

            rdma.wait()
            kept_bases.append(kept_base)
            ab = kept_base

        val = acc_ref[pl.ds((myslot - acc_base) * mc, mc), :]
        for t in range(NST):
            off = myslot - kept_bases[t]
            val = val + rbuf_ref[
                pl.ds((OFF[t] + off) * mc, mc), :
            ].astype(jnp.float32)
        out_ref[...] = val

    nch = sum(SZ)
    return pl.pallas_call(
        body,
        out_shape=jax.ShapeDtypeStruct((mc, n), jnp.float32),
        in_specs=[
            pl.BlockSpec(memory_space=pltpu.VMEM),
            pl.BlockSpec(memory_space=pltpu.VMEM),
        ],
        out_specs=pl.BlockSpec(memory_space=pltpu.VMEM),
        scratch_shapes=[
            pltpu.VMEM((m, k), jnp.bfloat16),
            pltpu.VMEM((k, n), jnp.bfloat16),
            pltpu.VMEM(((P // 2) * mc, n), jnp.float32),
            pltpu.VMEM((nch * mc, n), jnp.bfloat16),
            pltpu.VMEM((nch * mc, n), jnp.bfloat16),
            pltpu.SemaphoreType.DMA((NST,)),
            pltpu.SemaphoreType.DMA((NST,)),
        ],
        compiler_params=pltpu.CompilerParams(collective_id=0),
    )(A, B)
